# baseline (speedup 1.0000x reference)
import jax
import jax.numpy as jnp
from jax import lax
from jax.experimental import pallas as pl
from jax.experimental.pallas import tpu as pltpu
from jax.experimental.pallas import tpu_sc as plsc

NC = 2
NS = 16
NW = NC * NS


def _gather_body(cidx_hbm, mem_hbm, proto_hbm, gmem_hbm, gproto_hbm,
                 idx_v, membuf, protobuf, sem):
    c = lax.axis_index("c")
    s = lax.axis_index("s")
    w = s * NC + c
    per_w = idx_v.shape[0] * idx_v.shape[1]
    ch_n = idx_v.shape[0]
    base = w * per_w
    pltpu.sync_copy(cidx_hbm.at[w], idx_v)
    for ch in range(ch_n):
        row0 = base + ch * idx_v.shape[1]
        pltpu.async_copy(mem_hbm.at[idx_v.at[ch]], membuf, sem).wait()
        pltpu.sync_copy(membuf, gmem_hbm.at[pl.ds(row0, idx_v.shape[1])])
        pltpu.async_copy(proto_hbm.at[idx_v.at[ch]], protobuf, sem).wait()
        pltpu.sync_copy(protobuf, gproto_hbm.at[pl.ds(row0, idx_v.shape[1])])


def _sc_gather(c_idx, raw_memory, proto2d):
    twoB = c_idx.shape[0]
    N, D = raw_memory.shape
    KD = proto2d.shape[1]
    per_w = twoB // NW
    cidx3 = c_idx.reshape(NW, per_w // 128, 128)
    mesh = plsc.VectorSubcoreMesh(core_axis_name="c", subcore_axis_name="s")
    return pl.kernel(
        _gather_body,
        out_type=(jax.ShapeDtypeStruct((twoB, D), jnp.float32),
                  jax.ShapeDtypeStruct((twoB, KD), jnp.float32)),
        mesh=mesh,
        scratch_types=[
            pltpu.VMEM((per_w // 128, 128), jnp.int32),
            pltpu.VMEM((128, D), jnp.float32),
            pltpu.VMEM((128, KD), jnp.float32),
            pltpu.SemaphoreType.DMA,
        ],
    )(cidx3, raw_memory, proto2d)


def _ln(x, g, b):
    m = jnp.mean(x, axis=-1, keepdims=True)
    v = jnp.mean((x - m) ** 2, axis=-1, keepdims=True)
    return (x - m) / jnp.sqrt(v + 1e-5) * g + b


def _dense_body(edge_ref, time_ref, gmem_ref, gproto_ref, WeT_ref, be_ref,
                tw_ref, tb_ref, WqmT_ref, WqeT_ref, WqtT_ref, bq_ref,
                wgm_ref, wgc_ref, wgt_ref, bg_ref, temp_ref,
                lng_ref, lnb_ref, plng_ref, plnb_ref, u_ref):
    D = gmem_ref.shape[-1]
    K = gproto_ref.shape[-1] // D

    t = time_ref[...]
    te = jnp.cos(t * tw_ref[...] + tb_ref[...])

    ep = lax.dot(edge_ref[...], WeT_ref[...],
                 precision=lax.Precision.HIGHEST,
                 preferred_element_type=jnp.float32) + be_ref[...]
    nrm = jnp.sqrt(jnp.sum(ep * ep, axis=-1, keepdims=True))
    ep = ep / (nrm + 1e-8) * 10.0
    ep = jnp.clip(ep, -10.0, 10.0)

    ep_q = lax.dot(ep, WqeT_ref[...], precision=lax.Precision.HIGHEST,
                   preferred_element_type=jnp.float32)
    te_q = lax.dot(te, WqtT_ref[...], precision=lax.Precision.HIGHEST,
                   preferred_element_type=jnp.float32)
    te_g = jnp.sum(te * wgt_ref[...], axis=-1, keepdims=True)

    lng, lnb = lng_ref[...], lnb_ref[...]
    temp = jnp.clip(temp_ref[0, 0], 0.05, 2.0) + 1e-6

    for side in range(2):
        mem = gmem_ref[side]
        proto = gproto_ref[side]

        qi = (lax.dot(mem, WqmT_ref[...], precision=lax.Precision.HIGHEST,
                      preferred_element_type=jnp.float32)
              + ep_q + te_q + bq_ref[...])
        q = jnp.tanh(_ln(qi, lng, lnb))
        qn = q / jnp.maximum(
            jnp.sqrt(jnp.sum(q * q, axis=-1, keepdims=True)), 1e-12)

        sims = []
        pks = []
        for k in range(K):
            pk = _ln(proto[:, k * D:(k + 1) * D], plng_ref[...], plnb_ref[...])
            pks.append(pk)
            pn = pk / jnp.maximum(
                jnp.sqrt(jnp.sum(pk * pk, axis=-1, keepdims=True)), 1e-12)
            sims.append(jnp.sum(qn * pn, axis=-1, keepdims=True))
        sim = jnp.concatenate(sims, axis=-1) / temp
        sim = sim - jnp.max(sim, axis=-1, keepdims=True)
        e = jnp.exp(sim)
        attn = e / jnp.sum(e, axis=-1, keepdims=True)

        cand = attn[:, 0:1] * pks[0]
        for k in range(1, K):
            cand = cand + attn[:, k:k + 1] * pks[k]
        cand = jnp.clip(cand, -5.0, 5.0)

        g = (jnp.sum(jnp.clip(mem, -100.0, 100.0) * wgm_ref[...],
                     axis=-1, keepdims=True)
             + jnp.sum(cand * wgc_ref[...], axis=-1, keepdims=True)
             + te_g + bg_ref[0, 0])
        gate = 1.0 / (1.0 + jnp.exp(-g))

        upd = (1.0 - gate) * mem + gate * cand
        u_ref[side] = jnp.clip(_ln(upd, lng, lnb), -50.0, 50.0)


def _dense_phase(edge_features, time2d, gmem, gproto, WeT, be, tw, tb,
                 WqmT, WqeT, WqtT, bq, wgm, wgc, wgt, bg, temperature,
                 ln_g, ln_b, pln_g, pln_b, R=512):
    Bp = edge_features.shape[0]
    D = gmem.shape[-1]
    KD = gproto.shape[-1]
    grid = Bp // R
    full = lambda shape: pl.BlockSpec(shape, lambda i: (0,) * len(shape))
    return pl.pallas_call(
        _dense_body,
        grid=(grid,),
        in_specs=[
            pl.BlockSpec((R, edge_features.shape[1]), lambda i: (i, 0)),
            pl.BlockSpec((R, 1), lambda i: (i, 0)),
            pl.BlockSpec((2, R, D), lambda i: (0, i, 0)),
            pl.BlockSpec((2, R, KD), lambda i: (0, i, 0)),
            full((64, 128)), full((1, 128)), full((1, 64)), full((1, 64)),
            full((128, 128)), full((128, 128)), full((64, 128)),
            full((1, 128)), full((1, 128)), full((1, 128)), full((1, 64)),
            full((1, 1)), full((1, 1)),
            full((1, 128)), full((1, 128)), full((1, 128)), full((1, 128)),
        ],
        out_specs=pl.BlockSpec((2, R, D), lambda i: (0, i, 0)),
        out_shape=jax.ShapeDtypeStruct((2, Bp, D), jnp.float32),
    )(edge_features, time2d, gmem, gproto, WeT, be, tw, tb,
      WqmT, WqeT, WqtT, bq, wgm, wgc, wgt, bg, temperature,
      ln_g, ln_b, pln_g, pln_b)


def _scatter_body(raw_hbm, u_hbm, cidx_hbm, jg_hbm, out_hbm,
                  cidx_sc, cidx_sc2, jg_sc, lcid, lj, idxbuf, jbuf,
                  rowbuf, sem):
    N = raw_hbm.shape[0]
    c = lax.axis_index("c")
    s = lax.axis_index("s")
    half = N // NC
    half_lo = c * half
    n_copy_chunks = half // 200
    n_vregs = 2 * cidx_sc.shape[0] * (cidx_sc.shape[1] // 16)

    for k in range(8):
        cid = s + NS * k

        @pl.when(cid < n_copy_chunks)
        def _():
            r0 = half_lo + cid * 200
            pltpu.sync_copy(raw_hbm.at[pl.ds(r0, 200)],
                            out_hbm.at[pl.ds(r0, 200)])
    plsc.subcore_barrier()

    pltpu.sync_copy(cidx_hbm.at[s * 2], cidx_sc)
    pltpu.sync_copy(cidx_hbm.at[s * 2 + 1], cidx_sc2)
    pltpu.sync_copy(jg_hbm.at[s], jg_sc)

    def scan_body(i, off):
        r = i // 8
        col = (i % 8) * 16
        cv = jnp.where(r < 8, cidx_sc.at[r % 8][pl.ds(col, 16)],
                       cidx_sc2.at[r % 8][pl.ds(col, 16)])
        jv = jg_sc.at[r][pl.ds(col, 16)]
        m = (cv >= half_lo) & (cv < half_lo + half)
        cnt = jnp.sum(m.astype(jnp.int32), axis=0)
        plsc.store_compressed(lcid.at[pl.ds(off, 16)], cv, mask=m)
        plsc.store_compressed(lj.at[pl.ds(off, 16)], jv, mask=m)
        return off + cnt

    count = lax.fori_loop(0, n_vregs, scan_body, jnp.int32(0))

    @pl.when(count > 0)
    def _():
        zeros16 = jnp.zeros((16,), jnp.int32)
        bc_c = lcid[pl.ds(0, 16)][zeros16]
        bc_j = lj[pl.ds(0, 16)][zeros16]
        lanes = lax.iota(jnp.int32, 16)
        nch = (count + 127) // 128

        def sc_chunk(ch, _):
            for v in range(8):
                off = ch * 128 + v * 16
                valid = (off + lanes) < count
                cv = jnp.where(valid, lcid[pl.ds(off, 16)], bc_c)
                jv = jnp.where(valid, lj[pl.ds(off, 16)], bc_j)
                idxbuf[pl.ds(v * 16, 16)] = cv
                jbuf[pl.ds(v * 16, 16)] = jv
            pltpu.async_copy(u_hbm.at[jbuf], rowbuf, sem).wait()
            pltpu.async_copy(rowbuf, out_hbm.at[idxbuf], sem).wait()
            return 0

        lax.fori_loop(0, nch, sc_chunk, 0)


def _sc_scatter(raw_memory, u_flat, c_idx, jgather):
    N, D = raw_memory.shape
    twoB = c_idx.shape[0]
    per_s = twoB // NS
    cap = twoB + 16
    mesh = plsc.VectorSubcoreMesh(core_axis_name="c", subcore_axis_name="s")
    return pl.kernel(
        _scatter_body,
        out_type=jax.ShapeDtypeStruct((N, D), jnp.float32),
        mesh=mesh,
        compiler_params=pltpu.CompilerParams(needs_layout_passes=False),
        scratch_types=[
            pltpu.VMEM((per_s // 256, 128), jnp.int32),
            pltpu.VMEM((per_s // 256, 128), jnp.int32),
            pltpu.VMEM((per_s // 128, 128), jnp.int32),
            pltpu.VMEM((cap,), jnp.int32),
            pltpu.VMEM((cap,), jnp.int32),
            pltpu.VMEM((128,), jnp.int32),
            pltpu.VMEM((128,), jnp.int32),
            pltpu.VMEM((128, D), jnp.float32),
            pltpu.SemaphoreType.DMA,
        ],
    )(raw_memory, u_flat, c_idx.reshape(2 * NS, per_s // 256, 128),
      jgather.reshape(NS, per_s // 128, 128))


def kernel(source_nodes, target_nodes, edge_features, current_time,
           raw_memory, all_prototypes, We, be, tw, tb, Wq, bq, Wg, bg,
           temperature, ln_g, ln_b, pln_g, pln_b):
    N, D = raw_memory.shape
    B = source_nodes.shape[0]
    K = all_prototypes.shape[1]
    TD = tw.shape[0]

    c_idx = jnp.concatenate([source_nodes, target_nodes]).astype(jnp.int32)
    order = jnp.arange(2 * B, dtype=jnp.int32)
    ticket = jnp.zeros((N,), jnp.int32).at[c_idx].max(order)
    jgather = ticket[c_idx]

    proto2d = all_prototypes.reshape(N, K * D)
    gmem, gproto = _sc_gather(c_idx, raw_memory, proto2d)

    WeT = We.T
    WqmT = Wq[:, :D].T
    WqeT = Wq[:, D:2 * D].T
    WqtT = Wq[:, 2 * D:].T
    wgm = Wg[0, :D].reshape(1, D)
    wgc = Wg[0, D:2 * D].reshape(1, D)
    wgt = Wg[0, 2 * D:].reshape(1, TD)

    u = _dense_phase(
        edge_features, current_time.reshape(B, 1),
        gmem.reshape(2, B, D), gproto.reshape(2, B, K * D),
        WeT, be.reshape(1, D), tw.reshape(1, TD), tb.reshape(1, TD),
        WqmT, WqeT, WqtT, bq.reshape(1, D),
        wgm, wgc, wgt, bg.reshape(1, 1), temperature.reshape(1, 1),
        ln_g.reshape(1, D), ln_b.reshape(1, D),
        pln_g.reshape(1, D), pln_b.reshape(1, D))

    return _sc_scatter(raw_memory, u.reshape(2 * B, D), c_idx, jgather)


# speedup vs baseline: 1.0013x; 1.0013x over previous
import jax
import jax.numpy as jnp
from jax import lax
from jax.experimental import pallas as pl
from jax.experimental.pallas import tpu as pltpu
from jax.experimental.pallas import tpu_sc as plsc

NC = 2
NS = 16
NW = NC * NS


def _gather_body(cidx_hbm, mem_hbm, proto_hbm, gmem_hbm, gproto_hbm,
                 idx_v, membuf, protobuf, sem):
    c = lax.axis_index("c")
    s = lax.axis_index("s")
    w = s * NC + c
    per_w = idx_v.shape[0] * idx_v.shape[1]
    ch_n = idx_v.shape[0]
    base = w * per_w
    pltpu.sync_copy(cidx_hbm.at[w], idx_v)
    for ch in range(ch_n):
        row0 = base + ch * idx_v.shape[1]
        pltpu.async_copy(mem_hbm.at[idx_v.at[ch]], membuf, sem).wait()
        pltpu.sync_copy(membuf, gmem_hbm.at[pl.ds(row0, idx_v.shape[1])])
        pltpu.async_copy(proto_hbm.at[idx_v.at[ch]], protobuf, sem).wait()
        pltpu.sync_copy(protobuf, gproto_hbm.at[pl.ds(row0, idx_v.shape[1])])


def _sc_gather(c_idx, raw_memory, proto2d):
    twoB = c_idx.shape[0]
    N, D = raw_memory.shape
    KD = proto2d.shape[1]
    per_w = twoB // NW
    cidx3 = c_idx.reshape(NW, per_w // 128, 128)
    mesh = plsc.VectorSubcoreMesh(core_axis_name="c", subcore_axis_name="s")
    return pl.kernel(
        _gather_body,
        name="scgather",
        out_type=(jax.ShapeDtypeStruct((twoB, D), jnp.float32),
                  jax.ShapeDtypeStruct((twoB, KD), jnp.float32)),
        mesh=mesh,
        scratch_types=[
            pltpu.VMEM((per_w // 128, 128), jnp.int32),
            pltpu.VMEM((128, D), jnp.float32),
            pltpu.VMEM((128, KD), jnp.float32),
            pltpu.SemaphoreType.DMA,
        ],
    )(cidx3, raw_memory, proto2d)


def _ln(x, g, b):
    m = jnp.mean(x, axis=-1, keepdims=True)
    v = jnp.mean((x - m) ** 2, axis=-1, keepdims=True)
    return (x - m) / jnp.sqrt(v + 1e-5) * g + b


def _dense_body(edge_ref, time_ref, gmem_ref, gproto_ref, WeT_ref, be_ref,
                tw_ref, tb_ref, WqmT_ref, WqeT_ref, WqtT_ref, bq_ref,
                wgm_ref, wgc_ref, wgt_ref, bg_ref, temp_ref,
                lng_ref, lnb_ref, plng_ref, plnb_ref, u_ref):
    D = gmem_ref.shape[-1]
    K = gproto_ref.shape[-1] // D

    t = time_ref[...]
    te = jnp.cos(t * tw_ref[...] + tb_ref[...])

    ep = lax.dot(edge_ref[...], WeT_ref[...],
                 precision=lax.Precision.HIGHEST,
                 preferred_element_type=jnp.float32) + be_ref[...]
    nrm = jnp.sqrt(jnp.sum(ep * ep, axis=-1, keepdims=True))
    ep = ep / (nrm + 1e-8) * 10.0
    ep = jnp.clip(ep, -10.0, 10.0)

    ep_q = lax.dot(ep, WqeT_ref[...], precision=lax.Precision.HIGHEST,
                   preferred_element_type=jnp.float32)
    te_q = lax.dot(te, WqtT_ref[...], precision=lax.Precision.HIGHEST,
                   preferred_element_type=jnp.float32)
    te_g = jnp.sum(te * wgt_ref[...], axis=-1, keepdims=True)

    lng, lnb = lng_ref[...], lnb_ref[...]
    temp = jnp.clip(temp_ref[0, 0], 0.05, 2.0) + 1e-6

    for side in range(2):
        mem = gmem_ref[side]
        proto = gproto_ref[side]

        qi = (lax.dot(mem, WqmT_ref[...], precision=lax.Precision.HIGHEST,
                      preferred_element_type=jnp.float32)
              + ep_q + te_q + bq_ref[...])
        q = jnp.tanh(_ln(qi, lng, lnb))
        qn = q / jnp.maximum(
            jnp.sqrt(jnp.sum(q * q, axis=-1, keepdims=True)), 1e-12)

        sims = []
        pks = []
        for k in range(K):
            pk = _ln(proto[:, k * D:(k + 1) * D], plng_ref[...], plnb_ref[...])
            pks.append(pk)
            pn = pk / jnp.maximum(
                jnp.sqrt(jnp.sum(pk * pk, axis=-1, keepdims=True)), 1e-12)
            sims.append(jnp.sum(qn * pn, axis=-1, keepdims=True))
        sim = jnp.concatenate(sims, axis=-1) / temp
        sim = sim - jnp.max(sim, axis=-1, keepdims=True)
        e = jnp.exp(sim)
        attn = e / jnp.sum(e, axis=-1, keepdims=True)

        cand = attn[:, 0:1] * pks[0]
        for k in range(1, K):
            cand = cand + attn[:, k:k + 1] * pks[k]
        cand = jnp.clip(cand, -5.0, 5.0)

        g = (jnp.sum(jnp.clip(mem, -100.0, 100.0) * wgm_ref[...],
                     axis=-1, keepdims=True)
             + jnp.sum(cand * wgc_ref[...], axis=-1, keepdims=True)
             + te_g + bg_ref[0, 0])
        gate = 1.0 / (1.0 + jnp.exp(-g))

        upd = (1.0 - gate) * mem + gate * cand
        u_ref[side] = jnp.clip(_ln(upd, lng, lnb), -50.0, 50.0)


def _dense_phase(edge_features, time2d, gmem, gproto, WeT, be, tw, tb,
                 WqmT, WqeT, WqtT, bq, wgm, wgc, wgt, bg, temperature,
                 ln_g, ln_b, pln_g, pln_b, R=512):
    Bp = edge_features.shape[0]
    D = gmem.shape[-1]
    KD = gproto.shape[-1]
    grid = Bp // R
    full = lambda shape: pl.BlockSpec(shape, lambda i: (0,) * len(shape))
    return pl.pallas_call(
        _dense_body,
        grid=(grid,),
        in_specs=[
            pl.BlockSpec((R, edge_features.shape[1]), lambda i: (i, 0)),
            pl.BlockSpec((R, 1), lambda i: (i, 0)),
            pl.BlockSpec((2, R, D), lambda i: (0, i, 0)),
            pl.BlockSpec((2, R, KD), lambda i: (0, i, 0)),
            full((64, 128)), full((1, 128)), full((1, 64)), full((1, 64)),
            full((128, 128)), full((128, 128)), full((64, 128)),
            full((1, 128)), full((1, 128)), full((1, 128)), full((1, 64)),
            full((1, 1)), full((1, 1)),
            full((1, 128)), full((1, 128)), full((1, 128)), full((1, 128)),
        ],
        out_specs=pl.BlockSpec((2, R, D), lambda i: (0, i, 0)),
        out_shape=jax.ShapeDtypeStruct((2, Bp, D), jnp.float32),
    )(edge_features, time2d, gmem, gproto, WeT, be, tw, tb,
      WqmT, WqeT, WqtT, bq, wgm, wgc, wgt, bg, temperature,
      ln_g, ln_b, pln_g, pln_b)


def _scatter_body(raw_hbm, u_hbm, cidx_hbm, jg_hbm, out_hbm,
                  cidx_sc, cidx_sc2, jg_sc, lcid, lj, idxbuf, jbuf,
                  rowbuf, sem):
    N = raw_hbm.shape[0]
    c = lax.axis_index("c")
    s = lax.axis_index("s")
    half = N // NC
    half_lo = c * half
    n_copy_chunks = half // 200
    n_vregs = 2 * cidx_sc.shape[0] * (cidx_sc.shape[1] // 16)

    for k in range(8):
        cid = s + NS * k

        @pl.when(cid < n_copy_chunks)
        def _():
            r0 = half_lo + cid * 200
            pltpu.sync_copy(raw_hbm.at[pl.ds(r0, 200)],
                            out_hbm.at[pl.ds(r0, 200)])
    plsc.subcore_barrier()

    pltpu.sync_copy(cidx_hbm.at[s * 2], cidx_sc)
    pltpu.sync_copy(cidx_hbm.at[s * 2 + 1], cidx_sc2)
    pltpu.sync_copy(jg_hbm.at[s], jg_sc)

    def scan_body(i, off):
        r = i // 8
        col = (i % 8) * 16
        cv = jnp.where(r < 8, cidx_sc.at[r % 8][pl.ds(col, 16)],
                       cidx_sc2.at[r % 8][pl.ds(col, 16)])
        jv = jg_sc.at[r][pl.ds(col, 16)]
        m = (cv >= half_lo) & (cv < half_lo + half)
        cnt = jnp.sum(m.astype(jnp.int32), axis=0)
        plsc.store_compressed(lcid.at[pl.ds(off, 16)], cv, mask=m)
        plsc.store_compressed(lj.at[pl.ds(off, 16)], jv, mask=m)
        return off + cnt

    count = lax.fori_loop(0, n_vregs, scan_body, jnp.int32(0))

    @pl.when(count > 0)
    def _():
        zeros16 = jnp.zeros((16,), jnp.int32)
        bc_c = lcid[pl.ds(0, 16)][zeros16]
        bc_j = lj[pl.ds(0, 16)][zeros16]
        lanes = lax.iota(jnp.int32, 16)
        nch = (count + 127) // 128

        def sc_chunk(ch, _):
            for v in range(8):
                off = ch * 128 + v * 16
                valid = (off + lanes) < count
                cv = jnp.where(valid, lcid[pl.ds(off, 16)], bc_c)
                jv = jnp.where(valid, lj[pl.ds(off, 16)], bc_j)
                idxbuf[pl.ds(v * 16, 16)] = cv
                jbuf[pl.ds(v * 16, 16)] = jv
            pltpu.async_copy(u_hbm.at[jbuf], rowbuf, sem).wait()
            pltpu.async_copy(rowbuf, out_hbm.at[idxbuf], sem).wait()
            return 0

        lax.fori_loop(0, nch, sc_chunk, 0)


def _sc_scatter(raw_memory, u_flat, c_idx, jgather):
    N, D = raw_memory.shape
    twoB = c_idx.shape[0]
    per_s = twoB // NS
    cap = twoB + 16
    mesh = plsc.VectorSubcoreMesh(core_axis_name="c", subcore_axis_name="s")
    return pl.kernel(
        _scatter_body,
        name="scscatter",
        out_type=jax.ShapeDtypeStruct((N, D), jnp.float32),
        mesh=mesh,
        compiler_params=pltpu.CompilerParams(needs_layout_passes=False),
        scratch_types=[
            pltpu.VMEM((per_s // 256, 128), jnp.int32),
            pltpu.VMEM((per_s // 256, 128), jnp.int32),
            pltpu.VMEM((per_s // 128, 128), jnp.int32),
            pltpu.VMEM((cap,), jnp.int32),
            pltpu.VMEM((cap,), jnp.int32),
            pltpu.VMEM((128,), jnp.int32),
            pltpu.VMEM((128,), jnp.int32),
            pltpu.VMEM((128, D), jnp.float32),
            pltpu.SemaphoreType.DMA,
        ],
    )(raw_memory, u_flat, c_idx.reshape(2 * NS, per_s // 256, 128),
      jgather.reshape(NS, per_s // 128, 128))


def kernel(source_nodes, target_nodes, edge_features, current_time,
           raw_memory, all_prototypes, We, be, tw, tb, Wq, bq, Wg, bg,
           temperature, ln_g, ln_b, pln_g, pln_b):
    N, D = raw_memory.shape
    B = source_nodes.shape[0]
    K = all_prototypes.shape[1]
    TD = tw.shape[0]

    c_idx = jnp.concatenate([source_nodes, target_nodes]).astype(jnp.int32)
    order = jnp.arange(2 * B, dtype=jnp.int32)
    ticket = jnp.zeros((N,), jnp.int32).at[c_idx].max(order)
    jgather = ticket[c_idx]

    proto2d = all_prototypes.reshape(N, K * D)
    gmem, gproto = _sc_gather(c_idx, raw_memory, proto2d)

    WeT = We.T
    WqmT = Wq[:, :D].T
    WqeT = Wq[:, D:2 * D].T
    WqtT = Wq[:, 2 * D:].T
    wgm = Wg[0, :D].reshape(1, D)
    wgc = Wg[0, D:2 * D].reshape(1, D)
    wgt = Wg[0, 2 * D:].reshape(1, TD)

    u = _dense_phase(
        edge_features, current_time.reshape(B, 1),
        gmem.reshape(2, B, D), gproto.reshape(2, B, K * D),
        WeT, be.reshape(1, D), tw.reshape(1, TD), tb.reshape(1, TD),
        WqmT, WqeT, WqtT, bq.reshape(1, D),
        wgm, wgc, wgt, bg.reshape(1, 1), temperature.reshape(1, 1),
        ln_g.reshape(1, D), ln_b.reshape(1, D),
        pln_g.reshape(1, D), pln_b.reshape(1, D))

    return _sc_scatter(raw_memory, u.reshape(2 * B, D), c_idx, jgather)
